# baseline (speedup 1.0000x reference)
import jax
import jax.numpy as jnp
from jax import lax
from jax.experimental import pallas as pl
from jax.experimental.pallas import tpu as pltpu
from jax.experimental.pallas import tpu_sc as plsc

N = 10000
E = 320000
H = 128

NC = 2
NS = 16
NW = NC * NS
EP = E // NW
C = 80
NCHUNK = EP // C
NPAD = 10240
RPT = NPAD // NS
ZR = C


def _node_mlp_body(x_ref, w1_ref, b1_ref, u_ref):
    h = jnp.dot(x_ref[...], w1_ref[...], preferred_element_type=jnp.float32)
    h = h + b1_ref[...]
    u_ref[...] = h * 0.5 * (1.0 + lax.erf(h * 0.7071067811865476))


def _edge_body(u_hbm, src_hbm, dst_hbm, bases_hbm, out_hbm,
               srcall, d0, d1, u0, u1, b0, b1, ysh,
               sg0, sg1, sb0, sb1, sd0, sd1):
    c = lax.axis_index("c")
    s = lax.axis_index("s")
    w = c * NS + s
    base_e = w * EP

    pltpu.sync_copy(src_hbm.at[pl.ds(base_e, EP)], srcall)

    zero = jnp.zeros((16,), jnp.float32)

    def zstore(i, _):
        b0[i // 8, pl.ds((i % 8) * 16, 16)] = zero
        return 0

    lax.fori_loop(0, ZR * 8, zstore, 0)
    for j in range(RPT // ZR):
        pltpu.sync_copy(b0, ysh.at[pl.ds(s * RPT + j * ZR, ZR)])
    plsc.subcore_barrier()

    bufs = ((u0, b0, d0, sg0, sb0, sd0), (u1, b1, d1, sg1, sb1, sd1))

    def issue(g, u_v, b_v, d_v, sg, sb, sd):
        pltpu.async_copy(dst_hbm.at[pl.ds(base_e + g * C, C)], d_v, sd)
        pltpu.async_copy(u_hbm.at[srcall.at[pl.ds(g * C, C)]], u_v, sg)
        pltpu.async_copy(bases_hbm.at[pl.ds(base_e + g * C, C)], b_v, sb)

    def process(g, u_v, b_v, d_v, sg, sb, sd):
        pltpu.make_async_copy(dst_hbm.at[pl.ds(base_e + g * C, C)], d_v, sd).wait()
        pltpu.make_async_copy(u_hbm.at[srcall.at[pl.ds(g * C, C)]], u_v, sg).wait()
        pltpu.make_async_copy(bases_hbm.at[pl.ds(base_e + g * C, C)],
                              b_v, sb).wait()

        def mul_body(r, _):
            for k in range(4):
                w16 = u_v[r, pl.ds(k * 16, 16)]
                lo = lax.bitcast_convert_type(lax.shift_left(w16, 16),
                                              jnp.float32)
                hi = lax.bitcast_convert_type(
                    jnp.bitwise_and(w16, jnp.int32(-65536)), jnp.float32)
                sl0 = pl.ds(k * 32, 16)
                sl1 = pl.ds(k * 32 + 16, 16)
                b_v[r, sl0] = b_v[r, sl0] * lo
                b_v[r, sl1] = b_v[r, sl1] * hi
            return 0

        lax.fori_loop(0, C, mul_body, 0)
        pltpu.sync_copy(b_v, ysh.at[d_v], add=True)

    issue(0, *bufs[0])

    def pair(k, _):
        issue(2 * k + 1, *bufs[1])
        process(2 * k, *bufs[0])
        issue(2 * k + 2, *bufs[0])
        process(2 * k + 1, *bufs[1])
        return 0

    lax.fori_loop(0, (NCHUNK - 1) // 2, pair, 0)
    if NCHUNK % 2 == 0:
        issue(NCHUNK - 1, *bufs[1])
        process(NCHUNK - 2, *bufs[0])
        process(NCHUNK - 1, *bufs[1])
    else:
        process(NCHUNK - 1, *bufs[0])

    plsc.subcore_barrier()
    pltpu.sync_copy(ysh.at[pl.ds(s * RPT, RPT)],
                    out_hbm.at[c, pl.ds(s * RPT, RPT)])


def _ffn_body(x_ref, y0_ref, y1_ref, w2_ref, b2_ref, g1_ref, be1_ref,
              w3_ref, b3_ref, g2_ref, be2_ref, o_ref):
    eps = 1e-5
    x = x_ref[...] + y0_ref[...] + y1_ref[...]
    h = jnp.dot(x, w2_ref[...], preferred_element_type=jnp.float32)
    h = h + b2_ref[...]
    mean = jnp.mean(h, axis=0, keepdims=True)
    var = jnp.mean((h - mean) ** 2, axis=0, keepdims=True)
    h = (h - mean) * jax.lax.rsqrt(var + eps) * g1_ref[...] + be1_ref[...]
    h = jnp.maximum(h, 0.0)
    h = jnp.dot(h, w3_ref[...], preferred_element_type=jnp.float32)
    h = h + b3_ref[...]
    mean = jnp.mean(h, axis=0, keepdims=True)
    var = jnp.mean((h - mean) ** 2, axis=0, keepdims=True)
    h = (h - mean) * jax.lax.rsqrt(var + eps) * g2_ref[...] + be2_ref[...]
    h = jnp.maximum(h, 0.0)
    o_ref[...] = x + h


def kernel(x_feat, edge_index, bases, W1, b1, W2, b2, gamma1, beta1,
           W3, b3, gamma2, beta2):
    u = pl.pallas_call(
        _node_mlp_body,
        out_shape=jax.ShapeDtypeStruct((N, H), jnp.float32),
    )(x_feat, W1, b1.reshape(1, H))
    u_bf = u.reshape(N, H // 32, 2, 16).transpose(0, 1, 3, 2).astype(jnp.bfloat16)
    u_pk = lax.bitcast_convert_type(u_bf, jnp.int32).reshape(N, H // 2)

    src = edge_index[0]
    dst = edge_index[1]

    mesh = plsc.VectorSubcoreMesh(core_axis_name="c", subcore_axis_name="s")
    edge_kernel = pl.kernel(
        _edge_body,
        out_type=jax.ShapeDtypeStruct((NC, NPAD, H), jnp.float32),
        mesh=mesh,
        compiler_params=pltpu.CompilerParams(use_tc_tiling_on_sc=False),
        scratch_types=[
            pltpu.VMEM((EP,), jnp.int32),
            pltpu.VMEM((C,), jnp.int32),
            pltpu.VMEM((C,), jnp.int32),
            pltpu.VMEM((C, H // 2), jnp.int32),
            pltpu.VMEM((C, H // 2), jnp.int32),
            pltpu.VMEM((C, H), jnp.float32),
            pltpu.VMEM((C, H), jnp.float32),
            pltpu.VMEM_SHARED((NPAD, H), jnp.float32),
            pltpu.SemaphoreType.DMA,
            pltpu.SemaphoreType.DMA,
            pltpu.SemaphoreType.DMA,
            pltpu.SemaphoreType.DMA,
            pltpu.SemaphoreType.DMA,
            pltpu.SemaphoreType.DMA,
        ],
    )
    yp = edge_kernel(u_pk, src, dst, bases)

    out = pl.pallas_call(
        _ffn_body,
        out_shape=jax.ShapeDtypeStruct((N, H), jnp.float32),
    )(x_feat, yp[0, :N], yp[1, :N], W2, b2.reshape(1, H), gamma1.reshape(1, H),
      beta1.reshape(1, H), W3, b3.reshape(1, H), gamma2.reshape(1, H),
      beta2.reshape(1, H))
    return out


# speedup vs baseline: 1.6126x; 1.6126x over previous
import jax
import jax.numpy as jnp
from jax import lax
from jax.experimental import pallas as pl
from jax.experimental.pallas import tpu as pltpu
from jax.experimental.pallas import tpu_sc as plsc

N = 10000
E = 320000
H = 128

NC = 2
NS = 16
NW = NC * NS
EP = E // NW
C = 40
NCHUNK = EP // C
NPAD = 10240
RPT = NPAD // NS
ZR = C
NT = (NCHUNK - 5) // 3 + 1
assert NCHUNK == 3 * NT + 4


def _node_mlp_body(x_ref, w1_ref, b1_ref, u_ref):
    h = jnp.dot(x_ref[...], w1_ref[...], preferred_element_type=jnp.float32)
    h = h + b1_ref[...]
    u_ref[...] = h * 0.5 * (1.0 + lax.erf(h * 0.7071067811865476))


def _edge_body(u_hbm, src_hbm, dst_hbm, bases_hbm, out_hbm,
               srcall, d0, d1, d2, u0, u1, u2, b0, b1, b2, ysh,
               sg0, sg1, sg2, sb0, sb1, sb2, sd0, sd1, sd2,
               ss0, ss1, ss2):
    c = lax.axis_index("c")
    s = lax.axis_index("s")
    w = c * NS + s
    base_e = w * EP

    pltpu.sync_copy(src_hbm.at[pl.ds(base_e, EP)], srcall)

    zero = jnp.zeros((16,), jnp.float32)

    def zstore(i, _):
        b0[i // 8, pl.ds((i % 8) * 16, 16)] = zero
        return 0

    lax.fori_loop(0, ZR * 8, zstore, 0)
    for j in range(RPT // ZR):
        pltpu.sync_copy(b0, ysh.at[pl.ds(s * RPT + j * ZR, ZR)])
    plsc.subcore_barrier()

    bufs = ((u0, b0, d0, sg0, sb0, sd0, ss0),
            (u1, b1, d1, sg1, sb1, sd1, ss1),
            (u2, b2, d2, sg2, sb2, sd2, ss2))

    def issue(g, u_v, b_v, d_v, sg, sb, sd, ss, wait_scatter):
        if wait_scatter:
            pltpu.make_async_copy(b_v, ysh.at[d_v], ss).wait()
        pltpu.async_copy(dst_hbm.at[pl.ds(base_e + g * C, C)], d_v, sd)
        pltpu.async_copy(u_hbm.at[srcall.at[pl.ds(g * C, C)]], u_v, sg)
        pltpu.async_copy(bases_hbm.at[pl.ds(base_e + g * C, C)], b_v, sb)

    def process(g, u_v, b_v, d_v, sg, sb, sd, ss):
        pltpu.make_async_copy(dst_hbm.at[pl.ds(base_e + g * C, C)],
                              d_v, sd).wait()
        pltpu.make_async_copy(u_hbm.at[srcall.at[pl.ds(g * C, C)]],
                              u_v, sg).wait()
        pltpu.make_async_copy(bases_hbm.at[pl.ds(base_e + g * C, C)],
                              b_v, sb).wait()

        def mul_body(r, _):
            for k in range(8):
                sl = pl.ds(k * 16, 16)
                b_v[r, sl] = b_v[r, sl] * u_v[r, sl]
            return 0

        lax.fori_loop(0, C, mul_body, 0)
        pltpu.async_copy(b_v, ysh.at[d_v], ss, add=True)

    issue(0, *bufs[0], False)
    issue(1, *bufs[1], False)
    issue(2, *bufs[2], False)

    def ring(k, _):
        g = 3 * k
        process(g, *bufs[0])
        issue(g + 3, *bufs[0], True)
        process(g + 1, *bufs[1])
        issue(g + 4, *bufs[1], True)
        process(g + 2, *bufs[2])
        issue(g + 5, *bufs[2], True)
        return 0

    lax.fori_loop(0, NT, ring, 0)
    g = 3 * NT
    process(g, *bufs[0])
    issue(g + 3, *bufs[0], True)
    process(g + 1, *bufs[1])
    process(g + 2, *bufs[2])
    process(g + 3, *bufs[0])

    pltpu.make_async_copy(b1, ysh.at[d1], ss1).wait()
    pltpu.make_async_copy(b2, ysh.at[d2], ss2).wait()
    pltpu.make_async_copy(b0, ysh.at[d0], ss0).wait()

    plsc.subcore_barrier()
    pltpu.sync_copy(ysh.at[pl.ds(s * RPT, RPT)],
                    out_hbm.at[c, pl.ds(s * RPT, RPT)])


def _ffn_body(x_ref, y0_ref, y1_ref, w2_ref, b2_ref, g1_ref, be1_ref,
              w3_ref, b3_ref, g2_ref, be2_ref, o_ref):
    eps = 1e-5
    x = x_ref[...] + y0_ref[...] + y1_ref[...]
    h = jnp.dot(x, w2_ref[...], preferred_element_type=jnp.float32)
    h = h + b2_ref[...]
    mean = jnp.mean(h, axis=0, keepdims=True)
    var = jnp.mean((h - mean) ** 2, axis=0, keepdims=True)
    h = (h - mean) * jax.lax.rsqrt(var + eps) * g1_ref[...] + be1_ref[...]
    h = jnp.maximum(h, 0.0)
    h = jnp.dot(h, w3_ref[...], preferred_element_type=jnp.float32)
    h = h + b3_ref[...]
    mean = jnp.mean(h, axis=0, keepdims=True)
    var = jnp.mean((h - mean) ** 2, axis=0, keepdims=True)
    h = (h - mean) * jax.lax.rsqrt(var + eps) * g2_ref[...] + be2_ref[...]
    h = jnp.maximum(h, 0.0)
    o_ref[...] = x + h


def kernel(x_feat, edge_index, bases, W1, b1, W2, b2, gamma1, beta1,
           W3, b3, gamma2, beta2):
    u = pl.pallas_call(
        _node_mlp_body,
        out_shape=jax.ShapeDtypeStruct((N, H), jnp.float32),
    )(x_feat, W1, b1.reshape(1, H))

    src = edge_index[0]
    dst = edge_index[1]

    mesh = plsc.VectorSubcoreMesh(core_axis_name="c", subcore_axis_name="s")
    edge_kernel = pl.kernel(
        _edge_body,
        out_type=jax.ShapeDtypeStruct((NC, NPAD, H), jnp.float32),
        mesh=mesh,
        scratch_types=[
            pltpu.VMEM((EP,), jnp.int32),
            pltpu.VMEM((C,), jnp.int32),
            pltpu.VMEM((C,), jnp.int32),
            pltpu.VMEM((C,), jnp.int32),
            pltpu.VMEM((C, H), jnp.float32),
            pltpu.VMEM((C, H), jnp.float32),
            pltpu.VMEM((C, H), jnp.float32),
            pltpu.VMEM((C, H), jnp.float32),
            pltpu.VMEM((C, H), jnp.float32),
            pltpu.VMEM((C, H), jnp.float32),
            pltpu.VMEM_SHARED((NPAD, H), jnp.float32),
        ] + [pltpu.SemaphoreType.DMA] * 12,
    )
    yp = edge_kernel(u, src, dst, bases)

    out = pl.pallas_call(
        _ffn_body,
        out_shape=jax.ShapeDtypeStruct((N, H), jnp.float32),
    )(x_feat, yp[0, :N], yp[1, :N], W2, b2.reshape(1, H), gamma1.reshape(1, H),
      beta1.reshape(1, H), W3, b3.reshape(1, H), gamma2.reshape(1, H),
      beta2.reshape(1, H))
    return out


# speedup vs baseline: 1.6615x; 1.0304x over previous
import jax
import jax.numpy as jnp
from jax import lax
from jax.experimental import pallas as pl
from jax.experimental.pallas import tpu as pltpu
from jax.experimental.pallas import tpu_sc as plsc

N = 10000
E = 320000
H = 128

NC = 2
NS = 16
NW = NC * NS
EP = E // NW
C = 40
NCHUNK = EP // C
NPAD = 10240
RPT = NPAD // NS
ZR = C
NT = (NCHUNK - 5) // 3 + 1
assert NCHUNK == 3 * NT + 4


def _node_mlp_body(x_ref, w1_ref, b1_ref, u_ref):
    h = jnp.dot(x_ref[...], w1_ref[...], preferred_element_type=jnp.float32)
    h = h + b1_ref[...]
    u_ref[...] = h * 0.5 * (1.0 + lax.erf(h * 0.7071067811865476))


def _edge_body(u_hbm, src_hbm, dst_hbm, bases_hbm, out_hbm,
               srcall, d0, d1, d2, u0, u1, u2, b0, b1, b2, ysh,
               sg0, sg1, sg2, sb0, sb1, sb2, sd0, sd1, sd2,
               ss0, ss1, ss2):
    c = lax.axis_index("c")
    s = lax.axis_index("s")
    w = c * NS + s
    base_e = w * EP

    pltpu.sync_copy(src_hbm.at[pl.ds(base_e, EP)], srcall)

    zero = jnp.zeros((16,), jnp.float32)

    def zstore(i, _):
        b0[i // 8, pl.ds((i % 8) * 16, 16)] = zero
        return 0

    lax.fori_loop(0, ZR * 8, zstore, 0)
    for j in range(RPT // ZR):
        pltpu.sync_copy(b0, ysh.at[pl.ds(s * RPT + j * ZR, ZR)])
    plsc.subcore_barrier()

    bufs = ((u0, b0, d0, sg0, sb0, sd0, ss0),
            (u1, b1, d1, sg1, sb1, sd1, ss1),
            (u2, b2, d2, sg2, sb2, sd2, ss2))

    def issue(g, u_v, b_v, d_v, sg, sb, sd, ss, wait_scatter):
        if wait_scatter:
            pltpu.make_async_copy(b_v, ysh.at[d_v], ss).wait()
        pltpu.async_copy(dst_hbm.at[pl.ds(base_e + g * C, C)], d_v, sd)
        pltpu.async_copy(u_hbm.at[srcall.at[pl.ds(g * C, C)]], u_v, sg)
        pltpu.async_copy(bases_hbm.at[pl.ds(base_e + g * C, C)], b_v, sb)

    def process(g, u_v, b_v, d_v, sg, sb, sd, ss):
        pltpu.make_async_copy(dst_hbm.at[pl.ds(base_e + g * C, C)],
                              d_v, sd).wait()
        pltpu.make_async_copy(u_hbm.at[srcall.at[pl.ds(g * C, C)]],
                              u_v, sg).wait()
        pltpu.make_async_copy(bases_hbm.at[pl.ds(base_e + g * C, C)],
                              b_v, sb).wait()

        def mul_body(r, _):
            for k in range(8):
                sl = pl.ds(k * 16, 16)
                b_v[r, sl] = b_v[r, sl] * u_v[r, sl]
            return 0

        lax.fori_loop(0, C, mul_body, 0)
        pltpu.async_copy(b_v, ysh.at[d_v], ss, add=True)

    issue(0, *bufs[0], False)
    issue(1, *bufs[1], False)
    issue(2, *bufs[2], False)

    def ring(k, _):
        g = 3 * k
        process(g, *bufs[0])
        issue(g + 3, *bufs[0], True)
        process(g + 1, *bufs[1])
        issue(g + 4, *bufs[1], True)
        process(g + 2, *bufs[2])
        issue(g + 5, *bufs[2], True)
        return 0

    lax.fori_loop(0, NT, ring, 0)
    g = 3 * NT
    process(g, *bufs[0])
    issue(g + 3, *bufs[0], True)
    process(g + 1, *bufs[1])
    process(g + 2, *bufs[2])
    process(g + 3, *bufs[0])

    pltpu.make_async_copy(b1, ysh.at[d1], ss1).wait()
    pltpu.make_async_copy(b2, ysh.at[d2], ss2).wait()
    pltpu.make_async_copy(b0, ysh.at[d0], ss0).wait()

    plsc.subcore_barrier()
    pltpu.sync_copy(ysh.at[pl.ds(s * RPT, RPT)],
                    out_hbm.at[c, pl.ds(s * RPT, RPT)])


def _ffn_body(x_ref, yp_ref, w2_ref, b2_ref, g1_ref, be1_ref,
              w3_ref, b3_ref, g2_ref, be2_ref, o_ref):
    eps = 1e-5
    x = x_ref[...] + yp_ref[0, :N, :] + yp_ref[1, :N, :]
    h = jnp.dot(x, w2_ref[...], preferred_element_type=jnp.float32)
    h = h + b2_ref[...]
    mean = jnp.mean(h, axis=0, keepdims=True)
    var = jnp.mean((h - mean) ** 2, axis=0, keepdims=True)
    h = (h - mean) * jax.lax.rsqrt(var + eps) * g1_ref[...] + be1_ref[...]
    h = jnp.maximum(h, 0.0)
    h = jnp.dot(h, w3_ref[...], preferred_element_type=jnp.float32)
    h = h + b3_ref[...]
    mean = jnp.mean(h, axis=0, keepdims=True)
    var = jnp.mean((h - mean) ** 2, axis=0, keepdims=True)
    h = (h - mean) * jax.lax.rsqrt(var + eps) * g2_ref[...] + be2_ref[...]
    h = jnp.maximum(h, 0.0)
    o_ref[...] = x + h


def kernel(x_feat, edge_index, bases, W1, b1, W2, b2, gamma1, beta1,
           W3, b3, gamma2, beta2):
    u = pl.pallas_call(
        _node_mlp_body,
        out_shape=jax.ShapeDtypeStruct((N, H), jnp.float32),
    )(x_feat, W1, b1.reshape(1, H))

    src = edge_index[0]
    dst = edge_index[1]

    mesh = plsc.VectorSubcoreMesh(core_axis_name="c", subcore_axis_name="s")
    edge_kernel = pl.kernel(
        _edge_body,
        out_type=jax.ShapeDtypeStruct((NC, NPAD, H), jnp.float32),
        mesh=mesh,
        scratch_types=[
            pltpu.VMEM((EP,), jnp.int32),
            pltpu.VMEM((C,), jnp.int32),
            pltpu.VMEM((C,), jnp.int32),
            pltpu.VMEM((C,), jnp.int32),
            pltpu.VMEM((C, H), jnp.float32),
            pltpu.VMEM((C, H), jnp.float32),
            pltpu.VMEM((C, H), jnp.float32),
            pltpu.VMEM((C, H), jnp.float32),
            pltpu.VMEM((C, H), jnp.float32),
            pltpu.VMEM((C, H), jnp.float32),
            pltpu.VMEM_SHARED((NPAD, H), jnp.float32),
        ] + [pltpu.SemaphoreType.DMA] * 12,
    )
    yp = edge_kernel(u, src, dst, bases)

    out = pl.pallas_call(
        _ffn_body,
        out_shape=jax.ShapeDtypeStruct((N, H), jnp.float32),
    )(x_feat, yp, W2, b2.reshape(1, H), gamma1.reshape(1, H),
      beta1.reshape(1, H), W3, b3.reshape(1, H), gamma2.reshape(1, H),
      beta2.reshape(1, H))
    return out
